# baseline (speedup 1.0000x reference)
import functools

import jax
import jax.numpy as jnp
from jax.experimental import pallas as pl
from jax.experimental.pallas import tpu as pltpu


def _matmul_body(x_ref, w_ref, o_ref):
    o_ref[...] = jnp.dot(x_ref[...], w_ref[...],
                         preferred_element_type=jnp.float32)


def _matmul(x, w, block_m=1000):
    m, k = x.shape
    k2, n = w.shape
    assert k == k2
    return pl.pallas_call(
        _matmul_body,
        grid=(m // block_m,),
        in_specs=[
            pl.BlockSpec((block_m, k), lambda i: (i, 0)),
            pl.BlockSpec((k, n), lambda i: (0, 0)),
        ],
        out_specs=pl.BlockSpec((block_m, n), lambda i: (i, 0)),
        out_shape=jax.ShapeDtypeStruct((m, n), jnp.float32),
    )(x, w)


def _gat_conv(x, src, dst, W, a_src, a_dst, b):
    h = _matmul(x, W)
    alpha_s = h @ a_src
    alpha_d = h @ a_dst
    e = jax.nn.leaky_relu(alpha_s[src] + alpha_d[dst], negative_slope=0.2)
    n = x.shape[0]
    e_max = jax.ops.segment_max(e, dst, num_segments=n)
    e_max = jnp.where(jnp.isfinite(e_max), e_max, 0.0)
    e_exp = jnp.exp(e - e_max[dst])
    denom = jax.ops.segment_sum(e_exp, dst, num_segments=n)
    alpha = e_exp / (denom[dst] + 1e-16)
    out = jax.ops.segment_sum(alpha[:, None] * h[src], dst, num_segments=n)
    return out + b


def kernel(x, edge_index, edge_attr, batch, W1, a_src1, a_dst1, b1,
           W2, a_src2, a_dst2, b2, W_lin, b_lin):
    del edge_attr
    src = edge_index[0]
    dst = edge_index[1]
    h = _gat_conv(x, src, dst, W1, a_src1, a_dst1, b1)
    h = jax.nn.relu(h)
    h = _gat_conv(h, src, dst, W2, a_src2, a_dst2, b2)
    h = jax.nn.relu(h)
    B = 64
    pooled = jax.ops.segment_max(h, batch, num_segments=B)
    pooled = jnp.where(jnp.isfinite(pooled), pooled, 0.0)
    logits = pooled @ W_lin + b_lin
    return jax.nn.log_softmax(logits, axis=-1)


# speedup vs baseline: 1.3222x; 1.3222x over previous
import dataclasses
import functools

import jax
import jax.numpy as jnp
from jax import lax
from jax.experimental import pallas as pl
from jax.experimental.pallas import tpu as pltpu
from jax.experimental.pallas import tpu_sc as plsc

N = 10000
E = 160000
H = 512
NPAD = 10240
EPAD = 163840
EW = EPAD // 32
EB = EW // 128
NSLICE = NPAD // 16
FC = 128
NCHUNK = H // FC

_MESH = plsc.VectorSubcoreMesh(core_axis_name="c", subcore_axis_name="s")
_HIGH = lax.Precision.HIGHEST

_SC_CP = pltpu.CompilerParams()
if "needs_layout_passes" in pltpu.CompilerParams.__dataclass_fields__:
    _SC_CP = dataclasses.replace(_SC_CP, needs_layout_passes=False)


def _mm1_body(x_ref, w_ref, a_ref, h_ref, al_ref):
    h = jnp.dot(x_ref[...], w_ref[...], precision=_HIGH,
                preferred_element_type=jnp.float32)
    h_ref[...] = h
    al_ref[...] = jnp.dot(h, a_ref[...], precision=_HIGH,
                          preferred_element_type=jnp.float32)


def _mm_first(x, W, A):
    bm = 1000
    return pl.pallas_call(
        _mm1_body,
        grid=(N // bm,),
        in_specs=[
            pl.BlockSpec((bm, x.shape[1]), lambda i: (i, 0)),
            pl.BlockSpec(W.shape, lambda i: (0, 0)),
            pl.BlockSpec(A.shape, lambda i: (0, 0)),
        ],
        out_specs=[
            pl.BlockSpec((bm, H), lambda i: (i, 0)),
            pl.BlockSpec((bm, 2), lambda i: (i, 0)),
        ],
        out_shape=[
            jax.ShapeDtypeStruct((N, H), jnp.float32),
            jax.ShapeDtypeStruct((N, 2), jnp.float32),
        ],
    )(x, W, A)


def _mm2_body(op_ref, b_ref, w_ref, a_ref, h_ref, al_ref):
    x = jax.nn.relu(op_ref[0] + op_ref[1] + b_ref[...])
    h = jnp.dot(x, w_ref[...], precision=_HIGH,
                preferred_element_type=jnp.float32)
    h_ref[...] = h
    al_ref[...] = jnp.dot(h, a_ref[...], precision=_HIGH,
                          preferred_element_type=jnp.float32)


def _mm_second(op, b, W, A):
    bm = 1000
    return pl.pallas_call(
        _mm2_body,
        grid=(N // bm,),
        in_specs=[
            pl.BlockSpec((2, bm, H), lambda i: (0, i, 0)),
            pl.BlockSpec((H,), lambda i: (0,)),
            pl.BlockSpec(W.shape, lambda i: (0, 0)),
            pl.BlockSpec(A.shape, lambda i: (0, 0)),
        ],
        out_specs=[
            pl.BlockSpec((bm, H), lambda i: (i, 0)),
            pl.BlockSpec((bm, 2), lambda i: (i, 0)),
        ],
        out_shape=[
            jax.ShapeDtypeStruct((N, H), jnp.float32),
            jax.ShapeDtypeStruct((N, 2), jnp.float32),
        ],
    )(op, b, W, A)


def _combine_body(op_ref, b_ref, o_ref):
    o_ref[...] = jax.nn.relu(op_ref[0] + op_ref[1] + b_ref[...])


def _combine_relu(op, b):
    bm = 1000
    return pl.pallas_call(
        _combine_body,
        grid=(N // bm,),
        in_specs=[
            pl.BlockSpec((2, bm, H), lambda i: (0, i, 0)),
            pl.BlockSpec((H,), lambda i: (0,)),
        ],
        out_specs=pl.BlockSpec((bm, H), lambda i: (i, 0)),
        out_shape=jax.ShapeDtypeStruct((NPAD, H), jnp.float32),
    )(op, b)


def _sc_coeff(dst1d, w1d, den_part):
    @functools.partial(
        pl.kernel,
        mesh=_MESH,
        compiler_params=_SC_CP,
        out_type=jax.ShapeDtypeStruct((EPAD,), jnp.float32),
        scratch_types=[
            pltpu.VMEM((EW,), jnp.int32),
            pltpu.VMEM((EW,), jnp.float32),
            pltpu.VMEM((NPAD,), jnp.float32),
            pltpu.VMEM((NPAD,), jnp.float32),
        ],
    )
    def k(dst_hbm, w_hbm, den_hbm, cf_hbm, dst_v, cf_v, d0_v, d1_v):
        core = lax.axis_index("c")
        sub = lax.axis_index("s")
        ebase = (core * 16 + sub) * EW
        pltpu.sync_copy(dst_hbm.at[pl.ds(ebase, EW)], dst_v)
        pltpu.sync_copy(w_hbm.at[pl.ds(ebase, EW)], cf_v)
        pltpu.sync_copy(den_hbm.at[0], d0_v)
        pltpu.sync_copy(den_hbm.at[1], d1_v)

        @pl.loop(0, EW, step=16)
        def _(i):
            dv = dst_v[pl.ds(i, 16)]
            den = (plsc.load_gather(d0_v, [dv])
                   + plsc.load_gather(d1_v, [dv]) + jnp.float32(1e-16))
            cf_v[pl.ds(i, 16)] = cf_v[pl.ds(i, 16)] / den

        pltpu.sync_copy(cf_v, cf_hbm.at[pl.ds(ebase, EW)])

    return k(dst1d, w1d, den_part)


def _head_body(p_ref, w_ref, b_ref, o_ref):
    lg = jnp.dot(p_ref[...], w_ref[...], precision=_HIGH,
                 preferred_element_type=jnp.float32) + b_ref[...]
    m = jnp.max(lg, axis=-1, keepdims=True)
    s = jnp.log(jnp.sum(jnp.exp(lg - m), axis=-1, keepdims=True))
    o_ref[...] = lg - m - s


def _head(pooled, W_lin, b_lin):
    B, C = pooled.shape[0], W_lin.shape[1]
    return pl.pallas_call(
        _head_body,
        in_specs=[
            pl.BlockSpec(pooled.shape, lambda: (0, 0)),
            pl.BlockSpec(W_lin.shape, lambda: (0, 0)),
            pl.BlockSpec(b_lin.shape, lambda: (0,)),
        ],
        out_specs=pl.BlockSpec((B, C), lambda: (0, 0)),
        out_shape=jax.ShapeDtypeStruct((B, C), jnp.float32),
    )(pooled, W_lin, b_lin)


def _sc_pass1(src2d, dst2d, alpha_s, alpha_d):
    @functools.partial(
        pl.kernel,
        mesh=_MESH,
        compiler_params=_SC_CP,
        out_type=[
            jax.ShapeDtypeStruct((EPAD,), jnp.float32),
            jax.ShapeDtypeStruct((2, NPAD), jnp.float32),
        ],
        scratch_types=[
            pltpu.VMEM((EB, 128), jnp.int32),
            pltpu.VMEM((EB, 128), jnp.int32),
            pltpu.VMEM((EW,), jnp.float32),
            pltpu.VMEM((NPAD,), jnp.float32),
            pltpu.VMEM((NPAD,), jnp.float32),
            pltpu.VMEM((NPAD,), jnp.float32),
            pltpu.VMEM((NSLICE,), jnp.float32),
            pltpu.VMEM((NSLICE,), jnp.float32),
            pltpu.VMEM_SHARED((16, NPAD), jnp.float32),
        ],
    )
    def k(src_hbm, dst_hbm, as_hbm, ad_hbm, w_hbm, den_hbm,
          src_v, dst_v, w_v, as_v, ad_v, den_v, acc_v, tmp_v, sh_den):
        core = lax.axis_index("c")
        sub = lax.axis_index("s")
        wid = core * 16 + sub
        row0 = wid * EB

        pltpu.sync_copy(src_hbm.at[pl.ds(row0, EB)], src_v)
        pltpu.sync_copy(dst_hbm.at[pl.ds(row0, EB)], dst_v)
        pltpu.sync_copy(as_hbm, as_v)
        pltpu.sync_copy(ad_hbm, ad_v)
        ebase = wid * EW

        @pl.loop(0, NPAD, step=16)
        def _(i):
            den_v[pl.ds(i, 16)] = jnp.zeros((16,), jnp.float32)

        @pl.loop(0, EB)
        def _(bb):
            for j in range(8):
                sl = (bb, pl.ds(j * 16, 16))
                sv = src_v[sl]
                dv = dst_v[sl]
                av = plsc.load_gather(as_v, [sv])
                bv = plsc.load_gather(ad_v, [dv])
                e = av + bv
                e = jnp.where(e > 0, e, e * jnp.float32(0.2))
                wv = jnp.exp(e)
                w_v[pl.ds(bb * 128 + j * 16, 16)] = wv
                plsc.addupdate_scatter(den_v, [dv], wv)

        pltpu.sync_copy(w_v, w_hbm.at[pl.ds(ebase, EW)])

        pltpu.sync_copy(den_v, sh_den.at[sub])
        plsc.subcore_barrier()

        col0 = sub * NSLICE
        pltpu.sync_copy(sh_den.at[0, pl.ds(col0, NSLICE)], acc_v)
        for kk in range(1, 16):
            pltpu.sync_copy(sh_den.at[kk, pl.ds(col0, NSLICE)], tmp_v)

            @pl.loop(0, NSLICE, step=16)
            def _(i):
                acc_v[pl.ds(i, 16)] = acc_v[pl.ds(i, 16)] + tmp_v[pl.ds(i, 16)]

        pltpu.sync_copy(acc_v, den_hbm.at[core, pl.ds(col0, NSLICE)])

    return k(src2d, dst2d, alpha_s, alpha_d)


RN = 512
RNA = RN + 16
BT = 64
NB3 = EW // BT


def _sc_pass2(src1d, dst1d, cf1d, h):
    @functools.partial(
        pl.kernel,
        mesh=_MESH,
        compiler_params=_SC_CP,
        out_type=jax.ShapeDtypeStruct((2, NPAD, H), jnp.float32),
        scratch_types=[
            pltpu.VMEM((EW,), jnp.int32),
            pltpu.VMEM((EW,), jnp.int32),
            pltpu.VMEM((EW,), jnp.float32),
            pltpu.VMEM((EW + 16,), jnp.int32),
            pltpu.VMEM((NB3, BT), jnp.int32),
            pltpu.VMEM((NB3, BT), jnp.int32),
            pltpu.VMEM((1, BT), jnp.float32),
            pltpu.VMEM((BT, H), jnp.float32),
            pltpu.VMEM((BT, 128), jnp.float32),
            pltpu.VMEM((BT, 128), jnp.float32),
            pltpu.VMEM((BT, 128), jnp.float32),
            pltpu.VMEM((BT, 128), jnp.float32),
            pltpu.SemaphoreType.DMA,
            pltpu.VMEM_SHARED((RNA, 128), jnp.float32),
            pltpu.VMEM_SHARED((RNA, 128), jnp.float32),
            pltpu.VMEM_SHARED((RNA, 128), jnp.float32),
            pltpu.VMEM_SHARED((RNA, 128), jnp.float32),
        ],
    )
    def k(src_hbm, dst_hbm, cf_hbm, h_hbm, out_hbm,
          src_v, dst_v, cf_v, ids_v, gsrc_v, gdst_v, cfb_v, gbuf,
          sb0, sb1, sb2, sb3, gsem, sa0, sa1, sa2, sa3):
        core = lax.axis_index("c")
        sub = lax.axis_index("s")
        ebase = (core * 16 + sub) * EW
        sbufs = (sb0, sb1, sb2, sb3)
        saccs = (sa0, sa1, sa2, sa3)

        pltpu.sync_copy(src_hbm.at[pl.ds(ebase, EW)], src_v)
        pltpu.sync_copy(dst_hbm.at[pl.ds(ebase, EW)], dst_v)
        pltpu.sync_copy(cf_hbm.at[pl.ds(ebase, EW)], cf_v)

        @pl.loop(0, EW + 16, step=16)
        def _(i):
            ids_v[pl.ds(i, 16)] = jnp.zeros((16,), jnp.int32)

        iota = lax.iota(jnp.int32, 16)
        zrow = RN // 16

        @pl.loop(0, NPAD // RN)
        def _(r):
            r0 = r * RN
            r0v = jnp.full((16,), r0, jnp.int32)
            r1v = r0v + jnp.int32(RN)

            @pl.loop(0, BT)
            def _(rr):
                for j in range(128 // 16):
                    z16 = jnp.zeros((16,), jnp.float32)
                    sb0[rr, pl.ds(j * 16, 16)] = z16
                    sb1[rr, pl.ds(j * 16, 16)] = z16
                    sb2[rr, pl.ds(j * 16, 16)] = z16
                    sb3[rr, pl.ds(j * 16, 16)] = z16
            for c in range(4):
                pltpu.sync_copy(
                    sbufs[c].at[pl.ds(0, zrow)],
                    saccs[c].at[pl.ds(pl.multiple_of(sub * zrow, 16), zrow)])
            plsc.subcore_barrier()

            def comp(i, cnt):
                dv = dst_v[pl.ds(i * 16, 16)]
                m = jnp.logical_and(dv >= r0v, dv < r1v)
                plsc.store_compressed(ids_v.at[pl.ds(cnt, 16)],
                                      iota + i * 16, mask=m)
                return cnt + plsc.all_reduce_population_count(m)[0]

            cnt = lax.fori_loop(0, EW // 16, comp, jnp.int32(0))
            cntv = jnp.full((16,), cnt, jnp.int32)

            def prep(kb):
                kbc = jnp.minimum(kb, NB3 - 1)
                for g in range(BT // 16):
                    base = kbc * BT + g * 16
                    ids16 = ids_v[pl.ds(base, 16)]
                    valid = (iota + base) < cntv
                    sv = plsc.load_gather(src_v, [ids16])
                    dv = plsc.load_gather(dst_v, [ids16])
                    cf16 = plsc.load_gather(cf_v, [ids16])
                    gsrc_v[kbc, pl.ds(g * 16, 16)] = jnp.where(
                        valid, sv, jnp.int32(0))
                    gdst_v[kbc, pl.ds(g * 16, 16)] = jnp.where(
                        valid, dv - r0v, jnp.full((16,), RN, jnp.int32))
                    cfb_v[0, pl.ds(g * 16, 16)] = jnp.where(
                        valid, cf16, jnp.float32(0.0))

            def gstart(kb):
                kbc = jnp.minimum(kb, NB3 - 1)
                pltpu.make_async_copy(h_hbm.at[gsrc_v.at[kbc]], gbuf,
                                      gsem).start()

            prep(jnp.int32(0))
            gstart(jnp.int32(0))
            nb = (cnt + BT - 1) // BT

            def body(kb, _):
                pltpu.make_async_copy(h_hbm.at[pl.ds(0, BT)], gbuf,
                                      gsem).wait()

                @pl.loop(0, BT // 16)
                def _(g):
                    cfg = cfb_v[0, pl.ds(g * 16, 16)]
                    for rr in range(16):
                        cv = jnp.full((16,), cfg[rr], jnp.float32)
                        rw = g * 16 + rr
                        for c in range(4):
                            for j in range(128 // 16):
                                sbufs[c][rw, pl.ds(j * 16, 16)] = (
                                    gbuf[rw, pl.ds(c * 128 + j * 16, 16)]
                                    * cv)

                prep(kb + 1)
                gstart(kb + 1)
                kbc = jnp.minimum(kb, NB3 - 1)
                for c in range(4):
                    pltpu.sync_copy(sbufs[c], saccs[c].at[gdst_v.at[kbc]],
                                    add=True)
                return 0

            lax.fori_loop(0, nb, body, 0)
            pltpu.make_async_copy(h_hbm.at[pl.ds(0, BT)], gbuf, gsem).wait()
            plsc.subcore_barrier()

            for c in range(4):
                pltpu.sync_copy(
                    saccs[c].at[pl.ds(pl.multiple_of(sub * zrow, 16), zrow)],
                    out_hbm.at[core,
                               pl.ds(pl.multiple_of(r0 + sub * zrow, 16),
                                     zrow),
                               pl.ds(c * 128, 128)])
            plsc.subcore_barrier()

    return k(src1d, dst1d, cf1d, h)


def _sc_pool(h, batch_pad, num_graphs):
    @functools.partial(
        pl.kernel,
        mesh=_MESH,
        compiler_params=_SC_CP,
        out_type=jax.ShapeDtypeStruct((num_graphs, H), jnp.float32),
        scratch_types=[
            pltpu.VMEM((NPAD,), jnp.int32),
            pltpu.VMEM((32, H), jnp.float32),
            pltpu.VMEM((H,), jnp.float32),
        ],
    )
    def k(h_hbm, b_hbm, out_hbm, b_v, blk_v, acc_v):
        core = lax.axis_index("c")
        sub = lax.axis_index("s")
        wid = core * 16 + sub

        pltpu.sync_copy(b_hbm, b_v)

        g0 = wid * 2

        def count_below(carry, i):
            a0, a1, a2 = carry
            bv = b_v[pl.ds(i * 16, 16)]
            one = jnp.ones((16,), jnp.int32)
            zero = jnp.zeros((16,), jnp.int32)
            a0 = a0 + jnp.where(bv < g0, one, zero)
            a1 = a1 + jnp.where(bv < g0 + 1, one, zero)
            a2 = a2 + jnp.where(bv < g0 + 2, one, zero)
            return (a0, a1, a2)

        z16 = jnp.zeros((16,), jnp.int32)
        a0, a1, a2 = lax.fori_loop(0, NPAD // 16, lambda i, c: count_below(c, i),
                                   (z16, z16, z16))
        bounds = (jnp.sum(a0), jnp.sum(a1), jnp.sum(a2))

        for gi in range(2):
            start = bounds[gi]
            end = bounds[gi + 1]

            @pl.loop(0, H, step=16)
            def _(i):
                acc_v[pl.ds(i, 16)] = jnp.full((16,), -jnp.inf, jnp.float32)

            astart = (start // 8) * 8
            nblk = (end - astart + 31) // 32

            def blk_body(p, _):
                rs = pl.multiple_of(astart + p * 32, 8)
                pltpu.sync_copy(h_hbm.at[pl.ds(rs, 32)], blk_v)

                @pl.loop(0, 32)
                def _(rr):
                    row = rs + rr

                    @pl.when(jnp.logical_and(row >= start, row < end))
                    def _():
                        for j in range(H // 16):
                            sl = pl.ds(j * 16, 16)
                            acc_v[sl] = jnp.maximum(acc_v[sl], blk_v[rr, sl])

                return 0

            lax.fori_loop(0, nblk, blk_body, 0)

            @pl.loop(0, H, step=16)
            def _(i):
                v = acc_v[pl.ds(i, 16)]
                acc_v[pl.ds(i, 16)] = jnp.where(
                    v > -jnp.inf, v, jnp.zeros((16,), jnp.float32))

            pltpu.sync_copy(acc_v, out_hbm.at[g0 + gi])

    return k(h, batch_pad)


def _gat_layer(h, al, src2d, dst2d, src1d, dst1d):
    alpha_s = jnp.pad(al[:, 0], (0, NPAD - N))
    alpha_d = jnp.pad(al[:, 1], (0, NPAD - N))
    w1d, den_part = _sc_pass1(src2d, dst2d, alpha_s, alpha_d)
    cf = _sc_coeff(dst1d, w1d, den_part)
    return _sc_pass2(src1d, dst1d, cf, h)


def kernel(x, edge_index, edge_attr, batch, W1, a_src1, a_dst1, b1,
           W2, a_src2, a_dst2, b2, W_lin, b_lin):
    del edge_attr
    src1d = jnp.pad(edge_index[0], (0, EPAD - E))
    dst1d = jnp.pad(edge_index[1], (0, EPAD - E), constant_values=N)
    src = src1d.reshape(EPAD // 128, 128)
    dst = dst1d.reshape(EPAD // 128, 128)
    batch_pad = jnp.pad(batch, (0, NPAD - N), constant_values=64)

    A1 = jnp.stack([a_src1, a_dst1], axis=1)
    A2 = jnp.stack([a_src2, a_dst2], axis=1)

    h1, al = _mm_first(x, W1, A1)
    op1 = _gat_layer(h1, al, src, dst, src1d, dst1d)

    h2, al2 = _mm_second(op1, b1, W2, A2)
    op2 = _gat_layer(h2, al2, src, dst, src1d, dst1d)

    h2r = _combine_relu(op2, b2)
    pooled = _sc_pool(h2r, batch_pad, 64)
    return _head(pooled, W_lin, b_lin)


# speedup vs baseline: 5.1395x; 3.8870x over previous
import dataclasses
import functools

import jax
import jax.numpy as jnp
from jax import lax
from jax.experimental import pallas as pl
from jax.experimental.pallas import tpu as pltpu
from jax.experimental.pallas import tpu_sc as plsc

N = 10000
E = 160000
H = 512
NPAD = 10240
EPAD = 163840
EW = EPAD // 32
EB = EW // 128
NSLICE = NPAD // 16
FC = 128
NCHUNK = H // FC

_MESH = plsc.VectorSubcoreMesh(core_axis_name="c", subcore_axis_name="s")
_HIGH = lax.Precision.HIGHEST

_SC_CP = pltpu.CompilerParams()
if "needs_layout_passes" in pltpu.CompilerParams.__dataclass_fields__:
    _SC_CP = dataclasses.replace(_SC_CP, needs_layout_passes=False)


def _mm1_body(x_ref, w_ref, a_ref, h4_ref, al_ref):
    h = jnp.dot(x_ref[...], w_ref[...], precision=_HIGH,
                preferred_element_type=jnp.float32)
    for c in range(NCHUNK):
        h4_ref[c] = h[:, c * FC:(c + 1) * FC]
    al_ref[...] = jnp.dot(h, a_ref[...], precision=_HIGH,
                          preferred_element_type=jnp.float32)


def _mm_first(x, W, A):
    bm = 1000
    return pl.pallas_call(
        _mm1_body,
        grid=(N // bm,),
        in_specs=[
            pl.BlockSpec((bm, x.shape[1]), lambda i: (i, 0)),
            pl.BlockSpec(W.shape, lambda i: (0, 0)),
            pl.BlockSpec(A.shape, lambda i: (0, 0)),
        ],
        out_specs=[
            pl.BlockSpec((NCHUNK, bm, FC), lambda i: (0, i, 0)),
            pl.BlockSpec((bm, 2), lambda i: (i, 0)),
        ],
        out_shape=[
            jax.ShapeDtypeStruct((NCHUNK, N, FC), jnp.float32),
            jax.ShapeDtypeStruct((N, 2), jnp.float32),
        ],
    )(x, W, A)


def _mm2_body(op_ref, b_ref, w_ref, a_ref, h4_ref, al_ref):
    x = jax.nn.relu(op_ref[0] + op_ref[1] + b_ref[...])
    h = jnp.dot(x, w_ref[...], precision=_HIGH,
                preferred_element_type=jnp.float32)
    for c in range(NCHUNK):
        h4_ref[c] = h[:, c * FC:(c + 1) * FC]
    al_ref[...] = jnp.dot(h, a_ref[...], precision=_HIGH,
                          preferred_element_type=jnp.float32)


def _mm_second(op, b, W, A):
    bm = 1000
    return pl.pallas_call(
        _mm2_body,
        grid=(N // bm,),
        in_specs=[
            pl.BlockSpec((2, bm, H), lambda i: (0, i, 0)),
            pl.BlockSpec((H,), lambda i: (0,)),
            pl.BlockSpec(W.shape, lambda i: (0, 0)),
            pl.BlockSpec(A.shape, lambda i: (0, 0)),
        ],
        out_specs=[
            pl.BlockSpec((NCHUNK, bm, FC), lambda i: (0, i, 0)),
            pl.BlockSpec((bm, 2), lambda i: (i, 0)),
        ],
        out_shape=[
            jax.ShapeDtypeStruct((NCHUNK, N, FC), jnp.float32),
            jax.ShapeDtypeStruct((N, 2), jnp.float32),
        ],
    )(op, b, W, A)


def _combine_body(op_ref, b_ref, o_ref):
    o_ref[...] = jax.nn.relu(op_ref[0] + op_ref[1] + b_ref[...])


def _combine_relu(op, b):
    bm = 1000
    return pl.pallas_call(
        _combine_body,
        grid=(N // bm,),
        in_specs=[
            pl.BlockSpec((2, bm, H), lambda i: (0, i, 0)),
            pl.BlockSpec((H,), lambda i: (0,)),
        ],
        out_specs=pl.BlockSpec((bm, H), lambda i: (i, 0)),
        out_shape=jax.ShapeDtypeStruct((NPAD, H), jnp.float32),
    )(op, b)


def _den_sum_body(d_ref, o_ref):
    o_ref[...] = d_ref[0] + d_ref[1]


def _den_sum(den_part):
    return pl.pallas_call(
        _den_sum_body,
        in_specs=[pl.BlockSpec((2, NPAD), lambda: (0, 0))],
        out_specs=pl.BlockSpec((NPAD,), lambda: (0,)),
        out_shape=jax.ShapeDtypeStruct((NPAD,), jnp.float32),
    )(den_part)


def _head_body(p_ref, w_ref, b_ref, o_ref):
    lg = jnp.dot(p_ref[...], w_ref[...], precision=_HIGH,
                 preferred_element_type=jnp.float32) + b_ref[...]
    m = jnp.max(lg, axis=-1, keepdims=True)
    s = jnp.log(jnp.sum(jnp.exp(lg - m), axis=-1, keepdims=True))
    o_ref[...] = lg - m - s


def _head(pooled, W_lin, b_lin):
    B, C = pooled.shape[0], W_lin.shape[1]
    return pl.pallas_call(
        _head_body,
        in_specs=[
            pl.BlockSpec(pooled.shape, lambda: (0, 0)),
            pl.BlockSpec(W_lin.shape, lambda: (0, 0)),
            pl.BlockSpec(b_lin.shape, lambda: (0,)),
        ],
        out_specs=pl.BlockSpec((B, C), lambda: (0, 0)),
        out_shape=jax.ShapeDtypeStruct((B, C), jnp.float32),
    )(pooled, W_lin, b_lin)


def _sc_pass1(src2d, dst2d, alpha_s, alpha_d):
    @functools.partial(
        pl.kernel,
        mesh=_MESH,
        compiler_params=_SC_CP,
        out_type=[
            jax.ShapeDtypeStruct((EPAD,), jnp.float32),
            jax.ShapeDtypeStruct((2, NPAD), jnp.float32),
        ],
        scratch_types=[
            pltpu.VMEM((EB, 128), jnp.int32),
            pltpu.VMEM((EB, 128), jnp.int32),
            pltpu.VMEM((EW,), jnp.float32),
            pltpu.VMEM((NPAD,), jnp.float32),
            pltpu.VMEM((NPAD,), jnp.float32),
            pltpu.VMEM((NPAD,), jnp.float32),
            pltpu.VMEM((NSLICE,), jnp.float32),
            pltpu.VMEM((NSLICE,), jnp.float32),
            pltpu.VMEM_SHARED((16, NPAD), jnp.float32),
        ],
    )
    def k(src_hbm, dst_hbm, as_hbm, ad_hbm, w_hbm, den_hbm,
          src_v, dst_v, w_v, as_v, ad_v, den_v, acc_v, tmp_v, sh_den):
        core = lax.axis_index("c")
        sub = lax.axis_index("s")
        wid = core * 16 + sub
        row0 = wid * EB

        pltpu.sync_copy(src_hbm.at[pl.ds(row0, EB)], src_v)
        pltpu.sync_copy(dst_hbm.at[pl.ds(row0, EB)], dst_v)
        pltpu.sync_copy(as_hbm, as_v)
        pltpu.sync_copy(ad_hbm, ad_v)
        ebase = wid * EW

        @pl.loop(0, NPAD, step=16)
        def _(i):
            den_v[pl.ds(i, 16)] = jnp.zeros((16,), jnp.float32)

        @pl.loop(0, EB)
        def _(bb):
            for j in range(8):
                sl = (bb, pl.ds(j * 16, 16))
                sv = src_v[sl]
                dv = dst_v[sl]
                av = plsc.load_gather(as_v, [sv])
                bv = plsc.load_gather(ad_v, [dv])
                e = av + bv
                e = jnp.where(e > 0, e, e * jnp.float32(0.2))
                wv = jnp.exp(e)
                w_v[pl.ds(bb * 128 + j * 16, 16)] = wv
                plsc.addupdate_scatter(den_v, [dv], wv)

        pltpu.sync_copy(w_v, w_hbm.at[pl.ds(ebase, EW)])

        pltpu.sync_copy(den_v, sh_den.at[sub])
        plsc.subcore_barrier()

        col0 = sub * NSLICE
        pltpu.sync_copy(sh_den.at[0, pl.ds(col0, NSLICE)], acc_v)
        for kk in range(1, 16):
            pltpu.sync_copy(sh_den.at[kk, pl.ds(col0, NSLICE)], tmp_v)

            @pl.loop(0, NSLICE, step=16)
            def _(i):
                acc_v[pl.ds(i, 16)] = acc_v[pl.ds(i, 16)] + tmp_v[pl.ds(i, 16)]

        pltpu.sync_copy(acc_v, den_hbm.at[core, pl.ds(col0, NSLICE)])

    return k(src2d, dst2d, alpha_s, alpha_d)


ROWB = 64
NB2 = EW // ROWB


def _sc_pass2(src1d, dst64, w1d, den, h4flat):
    @functools.partial(
        pl.kernel,
        mesh=_MESH,
        compiler_params=_SC_CP,
        out_type=jax.ShapeDtypeStruct((2, NPAD, H), jnp.float32),
        scratch_types=[
            pltpu.VMEM((EW,), jnp.int32),
            pltpu.VMEM((NB2, ROWB), jnp.int32),
            pltpu.VMEM((EW,), jnp.float32),
            pltpu.VMEM((NPAD,), jnp.float32),
            pltpu.VMEM((ROWB, FC), jnp.float32),
            pltpu.VMEM((ROWB, FC), jnp.float32),
            pltpu.SemaphoreType.DMA,
            pltpu.SemaphoreType.DMA,
            pltpu.VMEM_SHARED((NPAD, FC), jnp.float32),
        ],
    )
    def k(src_hbm, dst_hbm, w_hbm, den_hbm, h4_hbm, out_hbm,
          src_v, dst_v, cf_v, den_v, row_a, row_b, gsem_a, gsem_b, sh_acc):
        core = lax.axis_index("c")
        sub = lax.axis_index("s")
        wid = core * 16 + sub
        ebase = wid * EW

        pltpu.sync_copy(src_hbm.at[pl.ds(ebase, EW)], src_v)
        pltpu.sync_copy(dst_hbm.at[pl.ds(wid * NB2, NB2)], dst_v)
        pltpu.sync_copy(w_hbm.at[pl.ds(ebase, EW)], cf_v)
        pltpu.sync_copy(den_hbm, den_v)

        @pl.loop(0, NB2)
        def _(bb):
            for j in range(ROWB // 16):
                dv = dst_v[bb, pl.ds(j * 16, 16)]
                dsum = plsc.load_gather(den_v, [dv]) + jnp.float32(1e-16)
                sl = pl.ds(bb * ROWB + j * 16, 16)
                cf_v[sl] = cf_v[sl] / dsum

        def gather_start(bb, row, sem):
            pltpu.make_async_copy(
                h4_hbm.at[src_v.at[pl.ds(bb * ROWB, ROWB)]], row, sem
            ).start()

        def gather_wait(row, sem):
            pltpu.make_async_copy(h4_hbm.at[pl.ds(0, ROWB)], row, sem).wait()

        def scale_scatter(bb, row):
            @pl.loop(0, ROWB // 16)
            def _(rg):
                cf16 = cf_v[pl.ds(bb * ROWB + rg * 16, 16)]
                for rr in range(16):
                    cv = jnp.full((16,), cf16[rr], jnp.float32)
                    r = rg * 16 + rr
                    for j in range(FC // 16):
                        sl = (r, pl.ds(j * 16, 16))
                        row[sl] = row[sl] * cv

            pltpu.sync_copy(row, sh_acc.at[dst_v.at[bb]], add=True)

        nz = NSLICE // ROWB
        for c in range(NCHUNK):
            if c > 0:
                @pl.loop(0, EW, step=16)
                def _(i):
                    src_v[pl.ds(i, 16)] = src_v[pl.ds(i, 16)] + jnp.int32(N)

            @pl.loop(0, ROWB)
            def _(r):
                for j in range(FC // 16):
                    row_a[r, pl.ds(j * 16, 16)] = jnp.zeros((16,), jnp.float32)

            for z in range(nz):
                pltpu.sync_copy(
                    row_a, sh_acc.at[pl.ds(sub * NSLICE + z * ROWB, ROWB)])
            plsc.subcore_barrier()

            gather_start(0, row_a, gsem_a)

            @pl.loop(0, NB2 // 2)
            def _(p):
                gather_start(2 * p + 1, row_b, gsem_b)
                gather_wait(row_a, gsem_a)
                scale_scatter(2 * p, row_a)

                @pl.when(p < NB2 // 2 - 1)
                def _():
                    gather_start(2 * p + 2, row_a, gsem_a)

                gather_wait(row_b, gsem_b)
                scale_scatter(2 * p + 1, row_b)

            plsc.subcore_barrier()
            nd = NSLICE // 128
            for z in range(nd):
                r0 = sub * NSLICE + z * 128
                pltpu.sync_copy(
                    sh_acc.at[pl.ds(r0, 128)],
                    out_hbm.at[core, pl.ds(r0, 128), pl.ds(c * FC, FC)])
            plsc.subcore_barrier()

    return k(src1d, dst64, w1d, den, h4flat)


def _sc_pool(h, batch_pad, num_graphs):
    @functools.partial(
        pl.kernel,
        mesh=_MESH,
        compiler_params=_SC_CP,
        out_type=jax.ShapeDtypeStruct((num_graphs, H), jnp.float32),
        scratch_types=[
            pltpu.VMEM((NPAD,), jnp.int32),
            pltpu.VMEM((32, H), jnp.float32),
            pltpu.VMEM((H,), jnp.float32),
        ],
    )
    def k(h_hbm, b_hbm, out_hbm, b_v, blk_v, acc_v):
        core = lax.axis_index("c")
        sub = lax.axis_index("s")
        wid = core * 16 + sub

        pltpu.sync_copy(b_hbm, b_v)

        g0 = wid * 2

        def count_below(carry, i):
            a0, a1, a2 = carry
            bv = b_v[pl.ds(i * 16, 16)]
            one = jnp.ones((16,), jnp.int32)
            zero = jnp.zeros((16,), jnp.int32)
            a0 = a0 + jnp.where(bv < g0, one, zero)
            a1 = a1 + jnp.where(bv < g0 + 1, one, zero)
            a2 = a2 + jnp.where(bv < g0 + 2, one, zero)
            return (a0, a1, a2)

        z16 = jnp.zeros((16,), jnp.int32)
        a0, a1, a2 = lax.fori_loop(0, NPAD // 16, lambda i, c: count_below(c, i),
                                   (z16, z16, z16))
        bounds = (jnp.sum(a0), jnp.sum(a1), jnp.sum(a2))

        for gi in range(2):
            start = bounds[gi]
            end = bounds[gi + 1]

            @pl.loop(0, H, step=16)
            def _(i):
                acc_v[pl.ds(i, 16)] = jnp.full((16,), -jnp.inf, jnp.float32)

            astart = (start // 8) * 8
            nblk = (end - astart + 31) // 32

            def blk_body(p, _):
                rs = pl.multiple_of(astart + p * 32, 8)
                pltpu.sync_copy(h_hbm.at[pl.ds(rs, 32)], blk_v)

                @pl.loop(0, 32)
                def _(rr):
                    row = rs + rr

                    @pl.when(jnp.logical_and(row >= start, row < end))
                    def _():
                        for j in range(H // 16):
                            sl = pl.ds(j * 16, 16)
                            acc_v[sl] = jnp.maximum(acc_v[sl], blk_v[rr, sl])

                return 0

            lax.fori_loop(0, nblk, blk_body, 0)

            @pl.loop(0, H, step=16)
            def _(i):
                v = acc_v[pl.ds(i, 16)]
                acc_v[pl.ds(i, 16)] = jnp.where(
                    v > -jnp.inf, v, jnp.zeros((16,), jnp.float32))

            pltpu.sync_copy(acc_v, out_hbm.at[g0 + gi])

    return k(h, batch_pad)


def _gat_layer(h4flat, al, src2d, dst2d, src1d, dst64):
    alpha_s = jnp.pad(al[:, 0], (0, NPAD - N))
    alpha_d = jnp.pad(al[:, 1], (0, NPAD - N))
    w1d, den_part = _sc_pass1(src2d, dst2d, alpha_s, alpha_d)
    den = _den_sum(den_part)
    return _sc_pass2(src1d, dst64, w1d, den, h4flat)


def kernel(x, edge_index, edge_attr, batch, W1, a_src1, a_dst1, b1,
           W2, a_src2, a_dst2, b2, W_lin, b_lin):
    del edge_attr
    src1d = jnp.pad(edge_index[0], (0, EPAD - E))
    dst1d = jnp.pad(edge_index[1], (0, EPAD - E), constant_values=N)
    src = src1d.reshape(EPAD // 128, 128)
    dst = dst1d.reshape(EPAD // 128, 128)
    dst64 = dst1d.reshape(EPAD // ROWB, ROWB)
    batch_pad = jnp.pad(batch, (0, NPAD - N), constant_values=64)

    A1 = jnp.stack([a_src1, a_dst1], axis=1)
    A2 = jnp.stack([a_src2, a_dst2], axis=1)

    h4, al = _mm_first(x, W1, A1)
    op1 = _gat_layer(h4.reshape(NCHUNK * N, FC), al, src, dst, src1d, dst64)

    h4b, al2 = _mm_second(op1, b1, W2, A2)
    op2 = _gat_layer(h4b.reshape(NCHUNK * N, FC), al2, src, dst, src1d, dst64)

    h2r = _combine_relu(op2, b2)
    pooled = _sc_pool(h2r, batch_pad, 64)
    return _head(pooled, W_lin, b_lin)
